# baseline (speedup 1.0000x reference)
import functools

import jax
import jax.numpy as jnp
import numpy as np
from jax.experimental import pallas as pl

_RC = 0.25
_S = 27
_CENTER = _S // 2
_BI = 8


def _shift_grid():
    r = np.array([-1, 0, 1])
    g = np.stack(np.meshgrid(r, r, r, indexing="ij"), axis=-1).reshape(-1, 3)
    return jnp.asarray(g, dtype=jnp.float32)


def _pair_kernel(posi_ref, base_ref, base3_ref, sod_ref, vec_ref, *, n, bi):
    ib = pl.program_id(1)
    ls = n * _S
    l3 = n * _S * 3

    posi = posi_ref[...]
    px = posi[:, 0:1]
    py = posi[:, 1:2]
    pz = posi[:, 2:3]

    dx = base_ref[0:1, :] - px
    dy = base_ref[1:2, :] - py
    dz = base_ref[2:3, :] - pz
    sod = dx * dx + dy * dy + dz * dz

    lane = jax.lax.broadcasted_iota(jnp.int32, (1, ls), 1)
    row = ib * bi + jax.lax.broadcasted_iota(jnp.int32, (bi, 1), 0)
    self_lane = row * _S + _CENTER
    mask = (sod < _RC * _RC) & (lane != self_lane)
    sod_ref[...] = jnp.where(mask, sod, 0.0)

    lane3 = jax.lax.broadcasted_iota(jnp.int32, (1, l3), 1)
    grp = lane3 // 3
    cm = lane3 - grp * 3
    pii = jnp.where(cm == 0, px, jnp.where(cm == 1, py, pz))
    v3 = base3_ref[0:1, :] - pii

    sq = v3 * v3
    rm1 = jnp.roll(sq, -1, 1)
    rm2 = jnp.roll(sq, -2, 1)
    rp1 = jnp.roll(sq, 1, 1)
    rp2 = jnp.roll(sq, 2, 1)
    s0 = sq + rm1 + rm2
    s1 = rp1 + sq + rm1
    s2 = rp2 + rp1 + sq
    sod3 = jnp.where(cm == 0, s0, jnp.where(cm == 1, s1, s2))

    mask3 = (sod3 < _RC * _RC) & (grp != self_lane)
    vec_ref[...] = jnp.where(mask3, v3, 0.0)


@jax.jit
def kernel(pos, cel, pbc, ent):
    B, N, _ = pos.shape
    f32 = jnp.float32
    sft = _shift_grid()
    sft_xyz = jnp.einsum("sc,bcd->bsd", sft, cel)
    valid = jnp.all(pbc[:, None, :] | (sft[None, :, :] == 0), axis=-1)

    s_off = (65536.0 * (jnp.arange(_S, dtype=f32) + 1.0))[None, :, None]
    sft_eff = sft_xyz + jnp.where(valid[..., None], 0.0, s_off)
    entf = (~ent).astype(f32)[..., None]
    posj_eff = pos + entf * 4096.0
    posi_eff = pos - entf * 16777216.0

    base = posj_eff[:, :, None, :] + sft_eff[:, None, :, :]
    base_t = jnp.stack(
        [base[..., c].reshape(B, N * _S) for c in range(3)], axis=1
    )
    base3 = base.reshape(B, 1, N * _S * 3)

    grid = (B, N // _BI)
    sod_flat, vec_flat = pl.pallas_call(
        functools.partial(_pair_kernel, n=N, bi=_BI),
        grid=grid,
        in_specs=[
            pl.BlockSpec((None, _BI, 3), lambda b, i: (b, i, 0)),
            pl.BlockSpec((None, 3, N * _S), lambda b, i: (b, 0, 0)),
            pl.BlockSpec((None, 1, N * _S * 3), lambda b, i: (b, 0, 0)),
        ],
        out_specs=[
            pl.BlockSpec((None, _BI, N * _S), lambda b, i: (b, i, 0)),
            pl.BlockSpec((None, _BI, N * _S * 3), lambda b, i: (b, i, 0)),
        ],
        out_shape=[
            jax.ShapeDtypeStruct((B, N, N * _S), f32),
            jax.ShapeDtypeStruct((B, N, N * _S * 3), f32),
        ],
    )(posi_eff, base_t, base3)

    vec = vec_flat.reshape(B, N, N, _S, 3)
    sod = sod_flat.reshape(B, N, N, _S)
    return vec, sod


# speedup vs baseline: 10.5396x; 10.5396x over previous
import functools

import jax
import jax.numpy as jnp
import numpy as np
from jax.experimental import pallas as pl

_RC = 0.25
_S = 27
_CENTER = _S // 2


def _shift_grid():
    r = np.array([-1, 0, 1])
    g = np.stack(np.meshgrid(r, r, r, indexing="ij"), axis=-1).reshape(-1, 3)
    return jnp.asarray(g, dtype=jnp.float32)


def _plane_kernel(row_ref, col_ref, vec_ref, sod_ref, *, n):
    s = pl.program_id(1)
    ii = jax.lax.broadcasted_iota(jnp.int32, (n, 1), 0)
    jj = jax.lax.broadcasted_iota(jnp.int32, (1, n), 1)

    dx = row_ref[0:1, :] - col_ref[:, 0:1]
    dy = row_ref[1:2, :] - col_ref[:, 1:2]
    dz = row_ref[2:3, :] - col_ref[:, 2:3]
    sod = dx * dx + dy * dy + dz * dz
    mask = (sod < _RC * _RC) & jnp.logical_or(ii != jj, s != _CENTER)

    zero = jnp.float32(0.0)
    vec_ref[0] = jnp.where(mask, dx, zero)
    vec_ref[1] = jnp.where(mask, dy, zero)
    vec_ref[2] = jnp.where(mask, dz, zero)
    sod_ref[...] = jnp.where(mask, sod, zero)


@jax.jit
def kernel(pos, cel, pbc, ent):
    B, N, _ = pos.shape
    f32 = jnp.float32
    sft = _shift_grid()
    sft_xyz = jnp.einsum("sc,bcd->bsd", sft, cel)
    valid = jnp.all(pbc[:, None, :] | (sft[None, :, :] == 0), axis=-1)

    s_off = (65536.0 * (jnp.arange(_S, dtype=f32) + 1.0))[None, :, None]
    sft_eff = sft_xyz + jnp.where(valid[..., None], 0.0, s_off)
    entf = (~ent).astype(f32)[..., None]
    posj_eff = pos + entf * 4096.0
    posi_eff = pos - entf * 16777216.0

    row_eff = (
        posj_eff.transpose(0, 2, 1)[:, None, :, :] + sft_eff[..., None]
    )

    grid = (B, _S)
    vec_t, sod_t = pl.pallas_call(
        functools.partial(_plane_kernel, n=N),
        grid=grid,
        in_specs=[
            pl.BlockSpec((None, None, 3, N), lambda b, s: (b, s, 0, 0)),
            pl.BlockSpec((None, N, 3), lambda b, s: (b, 0, 0)),
        ],
        out_specs=[
            pl.BlockSpec((None, None, 3, N, N), lambda b, s: (b, s, 0, 0, 0)),
            pl.BlockSpec((None, None, N, N), lambda b, s: (b, s, 0, 0)),
        ],
        out_shape=[
            jax.ShapeDtypeStruct((B, _S, 3, N, N), f32),
            jax.ShapeDtypeStruct((B, _S, N, N), f32),
        ],
    )(row_eff, posi_eff)

    vec = jnp.transpose(vec_t, (0, 3, 4, 1, 2))
    sod = jnp.transpose(sod_t, (0, 2, 3, 1))
    return vec, sod
